# baseline (speedup 1.0000x reference)
import functools
import math

import jax
import jax.numpy as jnp
from jax import lax
from jax.experimental import pallas as pl
from jax.experimental.pallas import tpu as pltpu
from jax.experimental.pallas import tpu_sc as plsc

D_MODEL = 64
SCALE = math.sqrt(D_MODEL)

NC = 2
NS = 16
NW = NC * NS

VOCAB = 1000000
XROWS = 4096
XCOLS = 200
ROWS_W = XROWS // NW
CH = (128, 72)
CHOFF = (0, 128)
NCHUNK = ROWS_W * 2
GRP = 4
NGRP = NCHUNK // GRP
NPAIR = NGRP // 2

_mesh = plsc.VectorSubcoreMesh(core_axis_name="c", subcore_axis_name="s")


COLS_BLK = 512
NBLK = -(-VOCAB // COLS_BLK)
VOCAB_PAD = NBLK * COLS_BLK


def _transpose_body(tt_ref, out_ref):
    blk = tt_ref[...]
    eye = jnp.eye(64, dtype=jnp.float32) * SCALE
    t = lax.dot_general(blk, eye, (((0,), (0,)), ((), ())),
                        preferred_element_type=jnp.float32)
    out_ref[...] = jnp.concatenate([t, t], axis=1)


_table_relayout = pl.pallas_call(
    _transpose_body,
    grid=(NBLK,),
    in_specs=[pl.BlockSpec((64, COLS_BLK), lambda i: (0, i))],
    out_specs=pl.BlockSpec((COLS_BLK, 128), lambda i: (i, 0)),
    out_shape=jax.ShapeDtypeStruct((VOCAB_PAD, 128), jnp.float32),
)


@functools.partial(
    pl.kernel,
    mesh=_mesh,
    compiler_params=pltpu.CompilerParams(use_tc_tiling_on_sc=False),
    out_type=jax.ShapeDtypeStruct((XROWS, XCOLS, 128), jnp.float32),
    scratch_types=[
        pltpu.VMEM((ROWS_W, XCOLS), jnp.int32),
        pltpu.VMEM((2, GRP, 128, D_MODEL), jnp.float32),
        pltpu.SemaphoreType.DMA,
        pltpu.SemaphoreType.DMA,
        pltpu.SemaphoreType.DMA,
        pltpu.SemaphoreType.DMA,
    ],
)
def _emb_lookup(idx_hbm, table_hbm, out_hbm, idx_v, rows_v,
                sem_g0, sem_g1, sem_s0, sem_s1):
    w = lax.axis_index("s") * NC + lax.axis_index("c")
    row0_w = w * ROWS_W
    pltpu.sync_copy(idx_hbm.at[pl.ds(row0_w, ROWS_W)], idx_v)

    tail_mul = jnp.where(
        lax.broadcasted_iota(jnp.int32, (16,), 0) < 8, 1, 2)

    def dbl_body(r, carry):
        for q in range(XCOLS // 16):
            sl = pl.ds(q * 16, 16)
            idx_v[r, sl] = idx_v[r, sl] * 2
        sl = pl.ds(XCOLS - 16, 16)
        idx_v[r, sl] = idx_v[r, sl] * tail_mul
        return carry

    lax.fori_loop(0, ROWS_W, dbl_body, 0, unroll=4)

    sems_g = (sem_g0, sem_g1)
    sems_s = (sem_s0, sem_s1)

    def gather_desc(g, p, i):
        rl = g * 2 + i // 2
        n, h = CH[i % 2], CHOFF[i % 2]
        return pltpu.make_async_copy(
            table_hbm.at[idx_v.at[rl, pl.ds(h, n)]],
            rows_v.at[p, i, pl.ds(0, n)], sems_g[p])

    def scatter_desc(g, p, i):
        rl = g * 2 + i // 2
        n, h = CH[i % 2], CHOFF[i % 2]
        return pltpu.make_async_copy(
            rows_v.at[p, i, pl.ds(0, n)],
            out_hbm.at[row0_w + rl, pl.ds(h, n), pl.ds(0, D_MODEL)],
            sems_s[p])

    for i in range(GRP):
        gather_desc(0, 0, i).start()

    def process(g, p, guard_prev, guard_next):
        def drain_prev():
            for i in range(GRP):
                scatter_desc(g - 1, 1 - p, i).wait()

        if guard_prev:
            pl.when(g >= 1)(drain_prev)
        else:
            drain_prev()

        def fire_next():
            for i in range(GRP):
                gather_desc(g + 1, 1 - p, i).start()

        if guard_next:
            pl.when(g <= NGRP - 2)(fire_next)
        else:
            fire_next()

        for i in range(GRP):
            gather_desc(g, p, i).wait()
        for i in range(GRP):
            scatter_desc(g, p, i).start()

    def pair_body(gp, carry):
        process(gp * 2, 0, guard_prev=True, guard_next=False)
        process(gp * 2 + 1, 1, guard_prev=False, guard_next=True)
        return carry

    lax.fori_loop(0, NPAIR, pair_body, 0)

    for i in range(GRP):
        scatter_desc(NGRP - 1, 1, i).wait()


def kernel(x, table):
    trm = _table_relayout(table.T)
    t64 = trm.reshape(2 * VOCAB_PAD, D_MODEL)
    op = _emb_lookup(x.astype(jnp.int32), t64)
    return lax.slice(op, (0, 0, 0), (XROWS, XCOLS, D_MODEL))


# speedup vs baseline: 1.3524x; 1.3524x over previous
import functools
import math

import jax
import jax.numpy as jnp
from jax import lax
from jax.experimental import pallas as pl
from jax.experimental.pallas import tpu as pltpu
from jax.experimental.pallas import tpu_sc as plsc

D_MODEL = 64
SCALE = math.sqrt(D_MODEL)

NC = 2
NS = 16
NW = NC * NS

XROWS = 4096
XCOLS = 200
ROWS_W = XROWS // NW
CH = (128, 72)
CHOFF = (0, 128)
NCHUNK = ROWS_W * 2
GRP = 4
NGRP = NCHUNK // GRP
NPAIR = NGRP // 2

_mesh = plsc.VectorSubcoreMesh(core_axis_name="c", subcore_axis_name="s")


@functools.partial(
    pl.kernel,
    mesh=_mesh,
    compiler_params=pltpu.CompilerParams(use_tc_tiling_on_sc=False),
    out_type=jax.ShapeDtypeStruct((XROWS, XCOLS, D_MODEL), jnp.float32),
    scratch_types=[
        pltpu.VMEM((ROWS_W, XCOLS), jnp.int32),
        pltpu.VMEM((2, GRP, 128, D_MODEL), jnp.float32),
        pltpu.SemaphoreType.DMA,
        pltpu.SemaphoreType.DMA,
        pltpu.SemaphoreType.DMA,
        pltpu.SemaphoreType.DMA,
    ],
)
def _emb_lookup(idx_hbm, table_hbm, out_hbm, idx_v, rows_v,
                sem_g0, sem_g1, sem_s0, sem_s1):
    w = lax.axis_index("s") * NC + lax.axis_index("c")
    row0_w = w * ROWS_W
    pltpu.sync_copy(idx_hbm.at[pl.ds(row0_w, ROWS_W)], idx_v)

    sems_g = (sem_g0, sem_g1)
    sems_s = (sem_s0, sem_s1)

    def gather_desc(g, p, i):
        rl = g * 2 + i // 2
        n, h = CH[i % 2], CHOFF[i % 2]
        return pltpu.make_async_copy(
            table_hbm.at[idx_v.at[rl, pl.ds(h, n)]],
            rows_v.at[p, i, pl.ds(0, n)], sems_g[p])

    def scatter_desc(g, p, i):
        rl = g * 2 + i // 2
        n, h = CH[i % 2], CHOFF[i % 2]
        return pltpu.make_async_copy(
            rows_v.at[p, i, pl.ds(0, n)],
            out_hbm.at[row0_w + rl, pl.ds(h, n)], sems_s[p])

    for i in range(GRP):
        gather_desc(0, 0, i).start()

    def process(g, p, guard_prev, guard_next):
        def drain_prev():
            for i in range(GRP):
                scatter_desc(g - 1, 1 - p, i).wait()

        if guard_prev:
            pl.when(g >= 1)(drain_prev)
        else:
            drain_prev()

        def fire_next():
            for i in range(GRP):
                gather_desc(g + 1, 1 - p, i).start()

        if guard_next:
            pl.when(g <= NGRP - 2)(fire_next)
        else:
            fire_next()

        for i in range(GRP):
            gather_desc(g, p, i).wait()
        for i in range(GRP):
            def row_body(r, c2):
                for q in range(D_MODEL // 16):
                    sl = pl.ds(q * 16, 16)
                    rows_v[p, i, r, sl] = rows_v[p, i, r, sl] * SCALE
                return c2

            lax.fori_loop(0, CH[i % 2], row_body, 0, unroll=4)
            scatter_desc(g, p, i).start()

    def pair_body(gp, carry):
        process(gp * 2, 0, guard_prev=True, guard_next=False)
        process(gp * 2 + 1, 1, guard_prev=False, guard_next=True)
        return carry

    lax.fori_loop(0, NPAIR, pair_body, 0)

    for i in range(GRP):
        scatter_desc(NGRP - 1, 1, i).wait()


def kernel(x, table):
    return _emb_lookup(x.astype(jnp.int32), table)
